# baseline (speedup 1.0000x reference)
import math

import jax
import jax.numpy as jnp
from jax.experimental import pallas as pl
from jax.experimental.pallas import tpu as pltpu

_TEMP = 0.07
_K = 10


def _rows_kernel(f_ref, fa_ref, ni_ref, na_ref, o_ref, *, R, N):
    i = pl.program_id(1)
    Fi = f_ref[0]
    Fa = fa_ref[0]
    dot = jax.lax.dot_general(Fi, Fa, (((1,), (1,)), ((), ())),
                              preferred_element_type=jnp.float32)
    den = jnp.maximum(ni_ref[0] * na_ref[0].T, 1e-8)
    sig = dot / den / _TEMP
    m = jnp.max(sig, axis=1, keepdims=True)
    p = jnp.exp(sig - m)
    sm = p / jnp.sum(p, axis=1, keepdims=True)
    rows = jax.lax.broadcasted_iota(jnp.int32, (R, N), 0) + i * R
    cols = jax.lax.broadcasted_iota(jnp.int32, (R, N), 1)
    smm = jnp.where(rows == cols, 0.0, sm)
    colf = cols.astype(jnp.float32)
    cur = smm
    for _ in range(_K):
        mx = jnp.max(cur, axis=1, keepdims=True)
        cand = jnp.where(cur == mx, colf, 2048.0)
        first = jnp.min(cand, axis=1, keepdims=True)
        cur = jnp.where(cand == first, -1.0, cur)
    o_ref[0] = jnp.where(cur < 0.0, smm, 0.0)


def _sym_kernel(qij_ref, qji_ref, o_ref):
    o_ref[0] = 0.5 * (qij_ref[0] + qji_ref[0].T)


def kernel(x, feature_logits, W):
    B, T, N, F = x.shape
    nbands = W.shape[0]
    L = max(8, T // 2)
    step = max(1, int(L * 0.5))

    starts = list(range(0, max(1, T - L + 1), step))
    segs = jnp.stack([x[:, s:s + L] for s in starts], axis=1)
    n = jnp.arange(L, dtype=jnp.float32)
    window = 0.5 * (1.0 - jnp.cos(2.0 * math.pi * n / L))
    segs = segs * window.reshape(1, 1, L, 1, 1)
    spec = jnp.fft.rfft(segs, axis=2)
    power = jnp.mean(jnp.abs(spec) ** 2, axis=1)
    feat_w = jax.nn.softmax(feature_logits, axis=0)
    power_agg = jnp.einsum('bfni,i->bfn', power, feat_w)
    feat = jnp.log(jnp.maximum(power_agg, 1e-8))
    feat = jnp.transpose(feat, (0, 2, 1))
    feat = feat @ W.T
    mu = jnp.mean(feat, axis=-1, keepdims=True)
    var = jnp.var(feat, axis=-1, keepdims=True)
    feat = (feat - mu) / jnp.sqrt(var + 1e-05)
    norms = jnp.linalg.norm(feat, axis=-1, keepdims=True)

    R = 256
    Q = pl.pallas_call(
        lambda *refs: _rows_kernel(*refs, R=R, N=N),
        grid=(B, N // R),
        in_specs=[
            pl.BlockSpec((1, R, nbands), lambda b, n: (b, n, 0)),
            pl.BlockSpec((1, N, nbands), lambda b, n: (b, 0, 0)),
            pl.BlockSpec((1, R, 1), lambda b, n: (b, n, 0)),
            pl.BlockSpec((1, N, 1), lambda b, n: (b, 0, 0)),
        ],
        out_specs=pl.BlockSpec((1, R, N), lambda b, n: (b, n, 0)),
        out_shape=jax.ShapeDtypeStruct((B, N, N), jnp.float32),
        compiler_params=pltpu.CompilerParams(
            dimension_semantics=("parallel", "parallel")),
    )(feat, feat, norms, norms)

    RO = 512
    A = pl.pallas_call(
        _sym_kernel,
        grid=(B, N // RO, N // RO),
        in_specs=[
            pl.BlockSpec((1, RO, RO), lambda b, i, j: (b, i, j)),
            pl.BlockSpec((1, RO, RO), lambda b, i, j: (b, j, i)),
        ],
        out_specs=pl.BlockSpec((1, RO, RO), lambda b, i, j: (b, i, j)),
        out_shape=jax.ShapeDtypeStruct((B, N, N), jnp.float32),
        compiler_params=pltpu.CompilerParams(
            dimension_semantics=("parallel", "parallel", "parallel")),
    )(Q, Q)
    return A


# speedup vs baseline: 1.0143x; 1.0143x over previous
import math

import jax
import jax.numpy as jnp
from jax.experimental import pallas as pl
from jax.experimental.pallas import tpu as pltpu

_TEMP = 0.07
_K = 10


def _rows_kernel(f_ref, fa_ref, ni_ref, na_ref, o_ref, *, R, N):
    i = pl.program_id(1)
    Fi = f_ref[0]
    Fa = fa_ref[0]
    dot = jax.lax.dot_general(Fi, Fa, (((1,), (1,)), ((), ())),
                              preferred_element_type=jnp.float32)
    den = jnp.maximum(ni_ref[0] * na_ref[0].T, 1e-8)
    sig = dot / den / _TEMP
    m = jnp.max(sig, axis=1, keepdims=True)
    p = jnp.exp(sig - m)
    sm = p / jnp.sum(p, axis=1, keepdims=True)
    rows = jax.lax.broadcasted_iota(jnp.int32, (R, N), 0) + i * R
    cols = jax.lax.broadcasted_iota(jnp.int32, (R, N), 1)
    smm = jnp.where(rows == cols, 0.0, sm)
    colf = cols.astype(jnp.float32)
    cur = smm
    for _ in range(_K):
        mx = jnp.max(cur, axis=1, keepdims=True)
        cand = jnp.where(cur == mx, colf, 2048.0)
        first = jnp.min(cand, axis=1, keepdims=True)
        cur = jnp.where(cand == first, -1.0, cur)
    o_ref[0] = jnp.where(cur < 0.0, smm, 0.0)


def _sym_kernel(qij_ref, qji_ref, o_ref):
    o_ref[0] = 0.5 * (qij_ref[0] + qji_ref[0].T)


def kernel(x, feature_logits, W):
    B, T, N, F = x.shape
    nbands = W.shape[0]
    L = max(8, T // 2)
    step = max(1, int(L * 0.5))

    starts = list(range(0, max(1, T - L + 1), step))
    segs = jnp.stack([x[:, s:s + L] for s in starts], axis=1)
    n = jnp.arange(L, dtype=jnp.float32)
    window = 0.5 * (1.0 - jnp.cos(2.0 * math.pi * n / L))
    segs = segs * window.reshape(1, 1, L, 1, 1)
    spec = jnp.fft.rfft(segs, axis=2)
    power = jnp.mean(jnp.abs(spec) ** 2, axis=1)
    feat_w = jax.nn.softmax(feature_logits, axis=0)
    power_agg = jnp.einsum('bfni,i->bfn', power, feat_w)
    feat = jnp.log(jnp.maximum(power_agg, 1e-8))
    feat = jnp.transpose(feat, (0, 2, 1))
    feat = feat @ W.T
    mu = jnp.mean(feat, axis=-1, keepdims=True)
    var = jnp.var(feat, axis=-1, keepdims=True)
    feat = (feat - mu) / jnp.sqrt(var + 1e-05)
    norms = jnp.linalg.norm(feat, axis=-1, keepdims=True)

    R = 512
    Q = pl.pallas_call(
        lambda *refs: _rows_kernel(*refs, R=R, N=N),
        grid=(B, N // R),
        in_specs=[
            pl.BlockSpec((1, R, nbands), lambda b, n: (b, n, 0)),
            pl.BlockSpec((1, N, nbands), lambda b, n: (b, 0, 0)),
            pl.BlockSpec((1, R, 1), lambda b, n: (b, n, 0)),
            pl.BlockSpec((1, N, 1), lambda b, n: (b, 0, 0)),
        ],
        out_specs=pl.BlockSpec((1, R, N), lambda b, n: (b, n, 0)),
        out_shape=jax.ShapeDtypeStruct((B, N, N), jnp.float32),
        compiler_params=pltpu.CompilerParams(
            dimension_semantics=("parallel", "parallel")),
    )(feat, feat, norms, norms)

    RO = 512
    A = pl.pallas_call(
        _sym_kernel,
        grid=(B, N // RO, N // RO),
        in_specs=[
            pl.BlockSpec((1, RO, RO), lambda b, i, j: (b, i, j)),
            pl.BlockSpec((1, RO, RO), lambda b, i, j: (b, j, i)),
        ],
        out_specs=pl.BlockSpec((1, RO, RO), lambda b, i, j: (b, i, j)),
        out_shape=jax.ShapeDtypeStruct((B, N, N), jnp.float32),
        compiler_params=pltpu.CompilerParams(
            dimension_semantics=("parallel", "parallel", "parallel")),
    )(Q, Q)
    return A


# speedup vs baseline: 1.0758x; 1.0607x over previous
import math

import jax
import jax.numpy as jnp
from jax.experimental import pallas as pl
from jax.experimental.pallas import tpu as pltpu

_TEMP = 0.07
_K = 10


def _graph_kernel(f_ref, n_ref, o_ref, *, N):
    F = f_ref[0]
    nr = n_ref[0]
    dot = jax.lax.dot_general(F, F, (((1,), (1,)), ((), ())),
                              preferred_element_type=jnp.float32)
    den = jnp.maximum(nr * nr.T, 1e-8)
    sig = dot / den / _TEMP
    m = jnp.max(sig, axis=1, keepdims=True)
    p = jnp.exp(sig - m)
    sm = p / jnp.sum(p, axis=1, keepdims=True)
    rows = jax.lax.broadcasted_iota(jnp.int32, (N, N), 0)
    cols = jax.lax.broadcasted_iota(jnp.int32, (N, N), 1)
    smm = jnp.where(rows == cols, 0.0, sm)
    colf = cols.astype(jnp.float32)
    cur = smm
    for _ in range(_K):
        mx = jnp.max(cur, axis=1, keepdims=True)
        cand = jnp.where(cur == mx, colf, 2048.0)
        first = jnp.min(cand, axis=1, keepdims=True)
        cur = jnp.where(cand == first, -1.0, cur)
    q = jnp.where(cur < 0.0, smm, 0.0)
    o_ref[0] = 0.5 * (q + q.T)


def kernel(x, feature_logits, W):
    B, T, N, F = x.shape
    nbands = W.shape[0]
    L = max(8, T // 2)
    step = max(1, int(L * 0.5))

    starts = list(range(0, max(1, T - L + 1), step))
    segs = jnp.stack([x[:, s:s + L] for s in starts], axis=1)
    n = jnp.arange(L, dtype=jnp.float32)
    window = 0.5 * (1.0 - jnp.cos(2.0 * math.pi * n / L))
    segs = segs * window.reshape(1, 1, L, 1, 1)
    spec = jnp.fft.rfft(segs, axis=2)
    power = jnp.mean(jnp.abs(spec) ** 2, axis=1)
    feat_w = jax.nn.softmax(feature_logits, axis=0)
    power_agg = jnp.einsum('bfni,i->bfn', power, feat_w)
    feat = jnp.log(jnp.maximum(power_agg, 1e-8))
    feat = jnp.transpose(feat, (0, 2, 1))
    feat = feat @ W.T
    mu = jnp.mean(feat, axis=-1, keepdims=True)
    var = jnp.var(feat, axis=-1, keepdims=True)
    feat = (feat - mu) / jnp.sqrt(var + 1e-05)
    norms = jnp.linalg.norm(feat, axis=-1, keepdims=True)

    A = pl.pallas_call(
        lambda *refs: _graph_kernel(*refs, N=N),
        grid=(B,),
        in_specs=[
            pl.BlockSpec((1, N, nbands), lambda b: (b, 0, 0)),
            pl.BlockSpec((1, N, 1), lambda b: (b, 0, 0)),
        ],
        out_specs=pl.BlockSpec((1, N, N), lambda b: (b, 0, 0)),
        out_shape=jax.ShapeDtypeStruct((B, N, N), jnp.float32),
        compiler_params=pltpu.CompilerParams(
            dimension_semantics=("parallel",)),
    )(feat, norms)
    return A
